# Initial kernel scaffold; baseline (speedup 1.0000x reference)
import jax
import jax.numpy as jnp
from jax.experimental import pallas as pl


def kernel(router_inputs, input_ids, W_router, b_router, embed_table, W1, b1, W2, b2, bn_gamma, bn_beta):
    raise NotImplementedError("write your pallas kernel here")


# speedup vs baseline: 5.0556x; 5.0556x over previous
import functools

import jax
import jax.numpy as jnp
from jax import lax
from jax.experimental import pallas as pl
from jax.experimental.pallas import tpu as pltpu
from jax.experimental.pallas import tpu_sc as plsc

_B, _S, _H, _V, _E, _F = 64, 128, 1024, 32000, 8, 2048
_CH = 64
_NC, _NS = 2, 16


def _fiota(shape, dim):
    return lax.broadcasted_iota(jnp.int32, shape, dim).astype(jnp.float32)


def _nt(a, b):
    return lax.dot_general(a, b, (((1,), (1,)), ((), ())),
                           preferred_element_type=jnp.float32)


def _router_body(x_ref, w_ref, wt_ref, b_ref, bt_ref,
                 perm_ref, es_ref, start_ref, ord_ref, nxe_ref, hn_ref):
    x = x_ref[...]
    logits = jnp.dot(x, w_ref[...], preferred_element_type=jnp.float32) + b_ref[...]
    logits_t = _nt(wt_ref[...], x) + bt_ref[...]
    col = _fiota((_B, _E), 1)
    mx = jnp.max(logits, axis=1, keepdims=True)
    chosen_c = jnp.min(jnp.where(logits >= mx, col, float(_E)), axis=1, keepdims=True)
    row = _fiota((_E, _B), 0)
    mx_t = jnp.max(logits_t, axis=0, keepdims=True)
    chosen_r = jnp.min(jnp.where(logits_t >= mx_t, row, float(_E)), axis=0, keepdims=True)
    ic = _fiota((_B, 1), 0)
    ir = _fiota((1, _B), 1)
    key_c = chosen_c * _B + ic
    key_r = chosen_r * _B + ir
    rank_r = jnp.sum(jnp.where(key_c < key_r, 1.0, 0.0), axis=0, keepdims=True)
    rank_c = jnp.sum(jnp.where(key_r < key_c, 1.0, 0.0), axis=1, keepdims=True)
    rr = _fiota((_B, _B), 0)
    cc = _fiota((_B, _B), 1)
    r_mat = jnp.where(jnp.broadcast_to(rank_r, (_B, _B)) == rr, 1.0, 0.0)
    perm_c = _nt(r_mat, ir)
    es_c = _nt(r_mat, chosen_r)
    q_mat = jnp.where(jnp.broadcast_to(rank_c, (_B, _B)) == cc, 1.0, 0.0)
    es_row = jnp.dot(chosen_r, q_mat, preferred_element_type=jnp.float32)
    shift = jnp.where(rr == cc + 1.0, 1.0, 0.0)
    prev_c = jnp.dot(shift, es_c + 1.0, preferred_element_type=jnp.float32)
    start_c = jnp.where(es_c + 1.0 != prev_c, 1.0, 0.0)
    shift_r = jnp.where(cc == rr + 1.0, 1.0, 0.0)
    prev_r = jnp.dot(es_row + 1.0, shift_r, preferred_element_type=jnp.float32)
    start_row = jnp.where(es_row + 1.0 != prev_r, 1.0, 0.0)
    lincl = jnp.where(rr >= cc, 1.0, 0.0)
    ord_c = jnp.dot(lincl, start_c, preferred_element_type=jnp.float32) - 1.0
    lincl_r = jnp.where(cc >= rr, 1.0, 0.0)
    ord_row = jnp.dot(start_row, lincl_r, preferred_element_type=jnp.float32) - 1.0
    n_mat = jnp.where(
        jnp.logical_and(jnp.broadcast_to(ord_row, (_B, _B)) == ord_c + 1.0,
                        jnp.broadcast_to(start_row, (_B, _B)) > 0.5),
        1.0, 0.0)
    nxe_c = _nt(n_mat, es_row)
    hn_c = jnp.sum(n_mat, axis=1, keepdims=True)
    perm_ref[...] = perm_c.astype(jnp.int32)
    es_ref[...] = es_c.astype(jnp.int32)
    start_ref[...] = start_c.astype(jnp.int32)
    ord_ref[...] = ord_c.astype(jnp.int32)
    nxe_ref[...] = nxe_c.astype(jnp.int32)
    hn_ref[...] = hn_c.astype(jnp.int32)


def _w_copies(w1_hbm, w2_hbm, e_idx, w1buf, w2buf, buf, sem1, sem2):
    c1 = pltpu.make_async_copy(w1_hbm.at[e_idx], w1buf.at[buf], sem1)
    c2 = pltpu.make_async_copy(w2_hbm.at[e_idx], w2buf.at[buf], sem2)
    return c1, c2


def _moe_body(perm_ref, es_ref, start_ref, ord_ref, nxe_ref, hn_ref,
              x_ref, b1_ref, b2_ref, w1_hbm, w2_hbm, out_ref,
              w1buf, w2buf, s1a, s2a, s1b, s2b):
    del perm_ref
    k = pl.program_id(0)
    par = ord_ref[k] % 2
    is_start = start_ref[k] == 1

    @pl.when(k == 0)
    def _prime():
        c1, c2 = _w_copies(w1_hbm, w2_hbm, es_ref[0], w1buf, w2buf, 0, s1a, s2a)
        c1.start()
        c2.start()

    @pl.when(jnp.logical_and(is_start, par == 0))
    def _flip0():
        c1, c2 = _w_copies(w1_hbm, w2_hbm, es_ref[k], w1buf, w2buf, 0, s1a, s2a)
        c1.wait()
        c2.wait()

        @pl.when(hn_ref[k] == 1)
        def _next():
            n1, n2 = _w_copies(w1_hbm, w2_hbm, nxe_ref[k], w1buf, w2buf, 1, s1b, s2b)
            n1.start()
            n2.start()

    @pl.when(jnp.logical_and(is_start, par == 1))
    def _flip1():
        c1, c2 = _w_copies(w1_hbm, w2_hbm, es_ref[k], w1buf, w2buf, 1, s1b, s2b)
        c1.wait()
        c2.wait()

        @pl.when(hn_ref[k] == 1)
        def _next():
            n1, n2 = _w_copies(w1_hbm, w2_hbm, nxe_ref[k], w1buf, w2buf, 0, s1a, s2a)
            n1.start()
            n2.start()

    h = jnp.dot(x_ref[0], w1buf[par], preferred_element_type=jnp.float32) + b1_ref[0]
    gm = jnp.mean(jax.nn.gelu(h), axis=0, keepdims=True)
    out_ref[0] = jnp.dot(gm, w2buf[par], preferred_element_type=jnp.float32) + b2_ref[0]


def _bn_body(p_ref, g_ref, bt_ref, out_ref):
    p = p_ref[...]
    mu = jnp.mean(p, axis=0, keepdims=True)
    d = p - mu
    var = jnp.mean(d * d, axis=0, keepdims=True)
    out_ref[...] = g_ref[...] * d * lax.rsqrt(var + 1e-5) + bt_ref[...]


def _make_gather_kernel():
    nw = _NC * _NS
    ntok = _B * _S
    per_w = ntok // nw
    mesh = plsc.VectorSubcoreMesh(core_axis_name="c", subcore_axis_name="s",
                                  num_cores=_NC, num_subcores=_NS)

    @functools.partial(
        pl.kernel,
        mesh=mesh,
        out_type=jax.ShapeDtypeStruct((ntok, _H), jnp.float32),
        scratch_types=[
            pltpu.VMEM((_CH,), jnp.int32),
            pltpu.VMEM((_CH, _H), jnp.float32),
            pltpu.SemaphoreType.DMA,
        ],
    )
    def _gather(table_hbm, ids_hbm, out_hbm, idx_v, rows_v, sem):
        wid = lax.axis_index("s") * _NC + lax.axis_index("c")
        base = wid * per_w
        for c in range(per_w // _CH):
            off = base + c * _CH
            pltpu.sync_copy(ids_hbm.at[pl.ds(off, _CH)], idx_v)
            pltpu.async_copy(table_hbm.at[idx_v], rows_v, sem).wait()
            pltpu.sync_copy(rows_v, out_hbm.at[pl.ds(off, _CH)])

    return _gather


_gather_cache = []


def _gather_rows(table, ids):
    if not _gather_cache:
        _gather_cache.append(_make_gather_kernel())
    return _gather_cache[0](table, ids)


def kernel(router_inputs, input_ids, W_router, b_router, embed_table, W1, b1, W2, b2, bn_gamma, bn_beta):
    routed = pl.pallas_call(
        _router_body,
        out_shape=[jax.ShapeDtypeStruct((_B, 1), jnp.int32)] * 6,
    )(router_inputs, W_router, W_router.T,
      b_router.reshape(1, _E), b_router.reshape(_E, 1))
    perm, es, start, ordn, nxe, hn = (r.reshape(_B) for r in routed)

    x = _gather_rows(embed_table, input_ids.reshape(-1))
    x = x.reshape(_B, _S, _H)

    pred = pl.pallas_call(
        _moe_body,
        grid_spec=pltpu.PrefetchScalarGridSpec(
            num_scalar_prefetch=6,
            grid=(_B,),
            in_specs=[
                pl.BlockSpec((1, _S, _H), lambda k, p, e, s, o, nx, h: (p[k], 0, 0)),
                pl.BlockSpec((1, 1, _F), lambda k, p, e, s, o, nx, h: (e[k], 0, 0)),
                pl.BlockSpec((1, 1, _H), lambda k, p, e, s, o, nx, h: (e[k], 0, 0)),
                pl.BlockSpec(memory_space=pl.ANY),
                pl.BlockSpec(memory_space=pl.ANY),
            ],
            out_specs=pl.BlockSpec((1, 1, _H), lambda k, p, e, s, o, nx, h: (p[k], 0, 0)),
            scratch_shapes=[
                pltpu.VMEM((2, _H, _F), jnp.float32),
                pltpu.VMEM((2, _F, _H), jnp.float32),
                pltpu.SemaphoreType.DMA,
                pltpu.SemaphoreType.DMA,
                pltpu.SemaphoreType.DMA,
                pltpu.SemaphoreType.DMA,
            ],
        ),
        out_shape=jax.ShapeDtypeStruct((_B, 1, _H), jnp.float32),
    )(perm, es, start, ordn, nxe, hn,
      x, b1.reshape(_E, 1, _F), b2.reshape(_E, 1, _H), W1, W2)
    pred = pred.reshape(_B, _H)

    out = pl.pallas_call(
        _bn_body,
        out_shape=jax.ShapeDtypeStruct((_B, _H), jnp.float32),
    )(pred, bn_gamma.reshape(1, _H), bn_beta.reshape(1, _H))
    return out


# speedup vs baseline: 5.7055x; 1.1286x over previous
import functools

import jax
import jax.numpy as jnp
from jax import lax
from jax.experimental import pallas as pl
from jax.experimental.pallas import tpu as pltpu
from jax.experimental.pallas import tpu_sc as plsc

_B, _S, _H, _V, _E, _F = 64, 128, 1024, 32000, 8, 2048
_CH = 64
_NC, _NS = 2, 16


def _fiota(shape, dim):
    return lax.broadcasted_iota(jnp.int32, shape, dim).astype(jnp.float32)


def _nt(a, b):
    return lax.dot_general(a, b, (((1,), (1,)), ((), ())),
                           preferred_element_type=jnp.float32)


def _router_body(x_ref, w_ref, wt_ref, b_ref, bt_ref,
                 perm_ref, es_ref, start_ref, ord_ref, nxe_ref, hn_ref,
                 ps_ref, cs_ref):
    x = x_ref[...]
    logits = jnp.dot(x, w_ref[...], preferred_element_type=jnp.float32) + b_ref[...]
    logits_t = _nt(wt_ref[...], x) + bt_ref[...]
    col = _fiota((_B, _E), 1)
    mx = jnp.max(logits, axis=1, keepdims=True)
    chosen_c = jnp.min(jnp.where(logits >= mx, col, float(_E)), axis=1, keepdims=True)
    row = _fiota((_E, _B), 0)
    mx_t = jnp.max(logits_t, axis=0, keepdims=True)
    chosen_r = jnp.min(jnp.where(logits_t >= mx_t, row, float(_E)), axis=0, keepdims=True)
    ic = _fiota((_B, 1), 0)
    ir = _fiota((1, _B), 1)
    key_c = chosen_c * _B + ic
    key_r = chosen_r * _B + ir
    rank_r = jnp.sum(jnp.where(key_c < key_r, 1.0, 0.0), axis=0, keepdims=True)
    rank_c = jnp.sum(jnp.where(key_r < key_c, 1.0, 0.0), axis=1, keepdims=True)
    rr = _fiota((_B, _B), 0)
    cc = _fiota((_B, _B), 1)
    r_mat = jnp.where(jnp.broadcast_to(rank_r, (_B, _B)) == rr, 1.0, 0.0)
    perm_c = _nt(r_mat, ir)
    es_c = _nt(r_mat, chosen_r)
    q_mat = jnp.where(jnp.broadcast_to(rank_c, (_B, _B)) == cc, 1.0, 0.0)
    es_row = jnp.dot(chosen_r, q_mat, preferred_element_type=jnp.float32)
    shift = jnp.where(rr == cc + 1.0, 1.0, 0.0)
    prev_c = jnp.dot(shift, es_c + 1.0, preferred_element_type=jnp.float32)
    start_c = jnp.where(es_c + 1.0 != prev_c, 1.0, 0.0)
    shift_r = jnp.where(cc == rr + 1.0, 1.0, 0.0)
    prev_r = jnp.dot(es_row + 1.0, shift_r, preferred_element_type=jnp.float32)
    start_row = jnp.where(es_row + 1.0 != prev_r, 1.0, 0.0)
    lincl = jnp.where(rr >= cc, 1.0, 0.0)
    ord_c = jnp.dot(lincl, start_c, preferred_element_type=jnp.float32) - 1.0
    lincl_r = jnp.where(cc >= rr, 1.0, 0.0)
    ord_row = jnp.dot(start_row, lincl_r, preferred_element_type=jnp.float32) - 1.0
    n_mat = jnp.where(
        jnp.logical_and(jnp.broadcast_to(ord_row, (_B, _B)) == ord_c + 1.0,
                        jnp.broadcast_to(start_row, (_B, _B)) > 0.5),
        1.0, 0.0)
    nxe_c = _nt(n_mat, es_row)
    hn_c = jnp.sum(n_mat, axis=1, keepdims=True)
    p_mat = jnp.where(
        jnp.logical_and(jnp.broadcast_to(ord_row, (_B, _B)) == ord_c - 1.0,
                        jnp.broadcast_to(start_row, (_B, _B)) > 0.5),
        1.0, 0.0)
    c_mat = jnp.where(
        jnp.logical_and(jnp.broadcast_to(ord_row, (_B, _B)) == ord_c,
                        jnp.broadcast_to(start_row, (_B, _B)) > 0.5),
        1.0, 0.0)
    ps_c = _nt(p_mat, ir)
    cs_c = _nt(c_mat, ir)
    perm_ref[...] = perm_c.astype(jnp.int32)
    es_ref[...] = es_c.astype(jnp.int32)
    start_ref[...] = start_c.astype(jnp.int32)
    ord_ref[...] = ord_c.astype(jnp.int32)
    nxe_ref[...] = nxe_c.astype(jnp.int32)
    hn_ref[...] = hn_c.astype(jnp.int32)
    ps_ref[...] = ps_c.astype(jnp.int32)
    cs_ref[...] = cs_c.astype(jnp.int32)


def _w_copies(w1_hbm, w2_hbm, e_idx, w1buf, w2buf, buf, sem1, sem2):
    c1 = pltpu.make_async_copy(w1_hbm.at[e_idx], w1buf.at[buf], sem1)
    c2 = pltpu.make_async_copy(w2_hbm.at[e_idx], w2buf.at[buf], sem2)
    return c1, c2


def _run_finalize(gm_scr, acc_ref, b2_ref, w2_e, lo, hi, e_idx):
    rows = lax.broadcasted_iota(jnp.int32, (_B, 1), 0)
    mask = jnp.logical_and(rows >= lo, rows < hi)
    gmm = jnp.where(mask, gm_scr[...].reshape(_B, _F), 0.0)
    contrib = jnp.dot(gmm, w2_e, preferred_element_type=jnp.float32)
    bias = jnp.where(mask, b2_ref[e_idx], 0.0)
    acc_ref[...] = acc_ref[...] + contrib + bias


def _moe_body(perm_ref, es_ref, start_ref, ord_ref, nxe_ref, hn_ref, ps_ref, cs_ref,
              x_ref, b1_ref, b2_ref, g_ref, bt_ref, w1_hbm, w2_hbm, out_ref,
              w1buf, w2buf, gm_scr, acc_ref, s1a, s2a, s1b, s2b):
    k = pl.program_id(0)
    par = ord_ref[k] % 2
    is_start = start_ref[k] == 1

    @pl.when(k == 0)
    def _prime():
        acc_ref[...] = jnp.zeros_like(acc_ref)
        c1, c2 = _w_copies(w1_hbm, w2_hbm, es_ref[0], w1buf, w2buf, 0, s1a, s2a)
        c1.start()
        c2.start()

    @pl.when(jnp.logical_and(is_start, par == 0))
    def _flip0():
        @pl.when(k > 0)
        def _fin():
            _run_finalize(gm_scr, acc_ref, b2_ref, w2buf[1],
                          ps_ref[k], k, es_ref[k - 1])

        c1, c2 = _w_copies(w1_hbm, w2_hbm, es_ref[k], w1buf, w2buf, 0, s1a, s2a)
        c1.wait()
        c2.wait()

        @pl.when(hn_ref[k] == 1)
        def _next():
            n1, n2 = _w_copies(w1_hbm, w2_hbm, nxe_ref[k], w1buf, w2buf, 1, s1b, s2b)
            n1.start()
            n2.start()

    @pl.when(jnp.logical_and(is_start, par == 1))
    def _flip1():
        @pl.when(k > 0)
        def _fin():
            _run_finalize(gm_scr, acc_ref, b2_ref, w2buf[0],
                          ps_ref[k], k, es_ref[k - 1])

        c1, c2 = _w_copies(w1_hbm, w2_hbm, es_ref[k], w1buf, w2buf, 1, s1b, s2b)
        c1.wait()
        c2.wait()

        @pl.when(hn_ref[k] == 1)
        def _next():
            n1, n2 = _w_copies(w1_hbm, w2_hbm, nxe_ref[k], w1buf, w2buf, 0, s1a, s2a)
            n1.start()
            n2.start()

    h = jnp.dot(x_ref[0], w1buf[par], preferred_element_type=jnp.float32) + b1_ref[0]
    gm_scr[k] = jnp.mean(jax.nn.gelu(h), axis=0, keepdims=True)

    @pl.when(k == _B - 1)
    def _last():
        _run_finalize(gm_scr, acc_ref, b2_ref, w2buf[par],
                      cs_ref[k], _B, es_ref[k])
        p = acc_ref[...]
        mu = jnp.mean(p, axis=0, keepdims=True)
        d = p - mu
        var = jnp.mean(d * d, axis=0, keepdims=True)
        normed = g_ref[...] * d * lax.rsqrt(var + 1e-5) + bt_ref[...]
        for i in range(_B):
            out_ref[perm_ref[i]] = normed[i:i + 1, :]


def _make_gather_kernel():
    nw = _NC * _NS
    ntok = _B * _S
    per_w = ntok // nw
    mesh = plsc.VectorSubcoreMesh(core_axis_name="c", subcore_axis_name="s",
                                  num_cores=_NC, num_subcores=_NS)

    @functools.partial(
        pl.kernel,
        mesh=mesh,
        out_type=jax.ShapeDtypeStruct((ntok, _H), jnp.float32),
        scratch_types=[
            pltpu.VMEM((_CH,), jnp.int32),
            pltpu.VMEM((_CH, _H), jnp.float32),
            pltpu.SemaphoreType.DMA,
        ],
    )
    def _gather(table_hbm, ids_hbm, out_hbm, idx_v, rows_v, sem):
        wid = lax.axis_index("s") * _NC + lax.axis_index("c")
        base = wid * per_w
        for c in range(per_w // _CH):
            off = base + c * _CH
            pltpu.sync_copy(ids_hbm.at[pl.ds(off, _CH)], idx_v)
            pltpu.async_copy(table_hbm.at[idx_v], rows_v, sem).wait()
            pltpu.sync_copy(rows_v, out_hbm.at[pl.ds(off, _CH)])

    return _gather


_gather_cache = []


def _gather_rows(table, ids):
    if not _gather_cache:
        _gather_cache.append(_make_gather_kernel())
    return _gather_cache[0](table, ids)


def kernel(router_inputs, input_ids, W_router, b_router, embed_table, W1, b1, W2, b2, bn_gamma, bn_beta):
    routed = pl.pallas_call(
        _router_body,
        out_shape=[jax.ShapeDtypeStruct((_B, 1), jnp.int32)] * 8,
    )(router_inputs, W_router, W_router.T,
      b_router.reshape(1, _E), b_router.reshape(_E, 1))
    perm, es, start, ordn, nxe, hn, ps, cs = (r.reshape(_B) for r in routed)

    x = _gather_rows(embed_table, input_ids.reshape(-1))
    x = x.reshape(_B, _S, _H)

    pred = pl.pallas_call(
        _moe_body,
        grid_spec=pltpu.PrefetchScalarGridSpec(
            num_scalar_prefetch=8,
            grid=(_B,),
            in_specs=[
                pl.BlockSpec((1, _S, _H),
                             lambda k, p, e, s, o, nx, h, pp, cc: (p[k], 0, 0)),
                pl.BlockSpec((1, 1, _F),
                             lambda k, p, e, s, o, nx, h, pp, cc: (e[k], 0, 0)),
                pl.BlockSpec((_E, 1, _H),
                             lambda k, p, e, s, o, nx, h, pp, cc: (0, 0, 0)),
                pl.BlockSpec((1, _H),
                             lambda k, p, e, s, o, nx, h, pp, cc: (0, 0)),
                pl.BlockSpec((1, _H),
                             lambda k, p, e, s, o, nx, h, pp, cc: (0, 0)),
                pl.BlockSpec(memory_space=pl.ANY),
                pl.BlockSpec(memory_space=pl.ANY),
            ],
            out_specs=pl.BlockSpec((_B, 1, _H),
                                   lambda k, p, e, s, o, nx, h, pp, cc: (0, 0, 0)),
            scratch_shapes=[
                pltpu.VMEM((2, _H, _F), jnp.float32),
                pltpu.VMEM((2, _F, _H), jnp.float32),
                pltpu.VMEM((_B, 1, _F), jnp.float32),
                pltpu.VMEM((_B, _H), jnp.float32),
                pltpu.SemaphoreType.DMA,
                pltpu.SemaphoreType.DMA,
                pltpu.SemaphoreType.DMA,
                pltpu.SemaphoreType.DMA,
            ],
        ),
        out_shape=jax.ShapeDtypeStruct((_B, 1, _H), jnp.float32),
    )(perm, es, start, ordn, nxe, hn, ps, cs,
      x, b1.reshape(_E, 1, _F), b2.reshape(_E, 1, _H),
      bn_gamma.reshape(1, _H), bn_beta.reshape(1, _H), W1, W2)
    return pred.reshape(_B, _H)


# speedup vs baseline: 5.7152x; 1.0017x over previous
import functools

import jax
import jax.numpy as jnp
from jax import lax
from jax.experimental import pallas as pl
from jax.experimental.pallas import tpu as pltpu
from jax.experimental.pallas import tpu_sc as plsc

_B, _S, _H, _V, _E, _F = 64, 128, 1024, 32000, 8, 2048
_CH = 64
_NC, _NS = 2, 16


def _fiota(shape, dim):
    return lax.broadcasted_iota(jnp.int32, shape, dim).astype(jnp.float32)


def _nt(a, b):
    return lax.dot_general(a, b, (((1,), (1,)), ((), ())),
                           preferred_element_type=jnp.float32)


def _router_body(x_ref, w_ref, wt_ref, b_ref, bt_ref,
                 perm_ref, es_ref, start_ref, ord_ref, nxe_ref, hn_ref,
                 ps_ref, cs_ref):
    x = x_ref[...]
    logits = jnp.dot(x, w_ref[...], preferred_element_type=jnp.float32) + b_ref[...]
    logits_t = _nt(wt_ref[...], x) + bt_ref[...]
    col = _fiota((_B, _E), 1)
    mx = jnp.max(logits, axis=1, keepdims=True)
    chosen_c = jnp.min(jnp.where(logits >= mx, col, float(_E)), axis=1, keepdims=True)
    row = _fiota((_E, _B), 0)
    mx_t = jnp.max(logits_t, axis=0, keepdims=True)
    chosen_r = jnp.min(jnp.where(logits_t >= mx_t, row, float(_E)), axis=0, keepdims=True)
    ic = _fiota((_B, 1), 0)
    ir = _fiota((1, _B), 1)
    key_c = chosen_c * _B + ic
    key_r = chosen_r * _B + ir
    rank_r = jnp.sum(jnp.where(key_c < key_r, 1.0, 0.0), axis=0, keepdims=True)
    rank_c = jnp.sum(jnp.where(key_r < key_c, 1.0, 0.0), axis=1, keepdims=True)
    rr = _fiota((_B, _B), 0)
    cc = _fiota((_B, _B), 1)
    r_mat = jnp.where(jnp.broadcast_to(rank_r, (_B, _B)) == rr, 1.0, 0.0)
    perm_c = _nt(r_mat, ir)
    es_c = _nt(r_mat, chosen_r)
    q_mat = jnp.where(jnp.broadcast_to(rank_c, (_B, _B)) == cc, 1.0, 0.0)
    es_row = jnp.dot(chosen_r, q_mat, preferred_element_type=jnp.float32)
    shift = jnp.where(rr == cc + 1.0, 1.0, 0.0)
    prev_c = jnp.dot(shift, es_c + 1.0, preferred_element_type=jnp.float32)
    start_c = jnp.where(es_c + 1.0 != prev_c, 1.0, 0.0)
    shift_r = jnp.where(cc == rr + 1.0, 1.0, 0.0)
    prev_r = jnp.dot(es_row + 1.0, shift_r, preferred_element_type=jnp.float32)
    start_row = jnp.where(es_row + 1.0 != prev_r, 1.0, 0.0)
    lincl = jnp.where(rr >= cc, 1.0, 0.0)
    ord_c = jnp.dot(lincl, start_c, preferred_element_type=jnp.float32) - 1.0
    lincl_r = jnp.where(cc >= rr, 1.0, 0.0)
    ord_row = jnp.dot(start_row, lincl_r, preferred_element_type=jnp.float32) - 1.0
    n_mat = jnp.where(
        jnp.logical_and(jnp.broadcast_to(ord_row, (_B, _B)) == ord_c + 1.0,
                        jnp.broadcast_to(start_row, (_B, _B)) > 0.5),
        1.0, 0.0)
    nxe_c = _nt(n_mat, es_row)
    hn_c = jnp.sum(n_mat, axis=1, keepdims=True)
    p_mat = jnp.where(
        jnp.logical_and(jnp.broadcast_to(ord_row, (_B, _B)) == ord_c - 1.0,
                        jnp.broadcast_to(start_row, (_B, _B)) > 0.5),
        1.0, 0.0)
    c_mat = jnp.where(
        jnp.logical_and(jnp.broadcast_to(ord_row, (_B, _B)) == ord_c,
                        jnp.broadcast_to(start_row, (_B, _B)) > 0.5),
        1.0, 0.0)
    ps_c = _nt(p_mat, ir)
    cs_c = _nt(c_mat, ir)
    perm_ref[...] = perm_c.astype(jnp.int32)
    es_ref[...] = es_c.astype(jnp.int32)
    start_ref[...] = start_c.astype(jnp.int32)
    ord_ref[...] = ord_c.astype(jnp.int32)
    nxe_ref[...] = nxe_c.astype(jnp.int32)
    hn_ref[...] = hn_c.astype(jnp.int32)
    ps_ref[...] = ps_c.astype(jnp.int32)
    cs_ref[...] = cs_c.astype(jnp.int32)


def _w_copies(w1_hbm, w2_hbm, e_idx, w1buf, w2buf, buf, sem1, sem2):
    c1 = pltpu.make_async_copy(w1_hbm.at[e_idx], w1buf.at[buf], sem1)
    c2 = pltpu.make_async_copy(w2_hbm.at[e_idx], w2buf.at[buf], sem2)
    return c1, c2


def _run_finalize(gm_scr, acc_ref, b2_ref, w2_e, lo, hi, e_idx):
    rows = lax.broadcasted_iota(jnp.int32, (_B, 1), 0)
    mask = jnp.logical_and(rows >= lo, rows < hi)
    gmm = jnp.where(mask, gm_scr[...].reshape(_B, _F), 0.0)
    contrib = jnp.dot(gmm, w2_e, preferred_element_type=jnp.float32)
    bias = jnp.where(mask, b2_ref[e_idx], 0.0)
    acc_ref[...] = acc_ref[...] + contrib + bias


def _moe_body(perm_ref, es_ref, start_ref, ord_ref, nxe_ref, hn_ref, ps_ref, cs_ref,
              x_ref, b1_ref, b2_ref, g_ref, bt_ref, w1_hbm, w2_hbm, out_ref,
              w1buf, w2buf, gm_scr, acc_ref, s1a, s2a, s1b, s2b):
    k = pl.program_id(0)
    par = ord_ref[k] % 2
    is_start = start_ref[k] == 1

    @pl.when(k == 0)
    def _prime():
        acc_ref[...] = jnp.zeros_like(acc_ref)
        c1, c2 = _w_copies(w1_hbm, w2_hbm, es_ref[0], w1buf, w2buf, 0, s1a, s2a)
        c1.start()
        c2.start()

    @pl.when(jnp.logical_and(is_start, par == 0))
    def _flip0():
        @pl.when(k > 0)
        def _fin():
            _run_finalize(gm_scr, acc_ref, b2_ref, w2buf[1],
                          ps_ref[k], k, es_ref[k - 1])

        c1, c2 = _w_copies(w1_hbm, w2_hbm, es_ref[k], w1buf, w2buf, 0, s1a, s2a)
        c1.wait()
        c2.wait()

        @pl.when(hn_ref[k] == 1)
        def _next():
            n1, n2 = _w_copies(w1_hbm, w2_hbm, nxe_ref[k], w1buf, w2buf, 1, s1b, s2b)
            n1.start()
            n2.start()

    @pl.when(jnp.logical_and(is_start, par == 1))
    def _flip1():
        @pl.when(k > 0)
        def _fin():
            _run_finalize(gm_scr, acc_ref, b2_ref, w2buf[0],
                          ps_ref[k], k, es_ref[k - 1])

        c1, c2 = _w_copies(w1_hbm, w2_hbm, es_ref[k], w1buf, w2buf, 1, s1b, s2b)
        c1.wait()
        c2.wait()

        @pl.when(hn_ref[k] == 1)
        def _next():
            n1, n2 = _w_copies(w1_hbm, w2_hbm, nxe_ref[k], w1buf, w2buf, 0, s1a, s2a)
            n1.start()
            n2.start()

    h = jnp.dot(x_ref[0], w1buf[par], preferred_element_type=jnp.float32) + b1_ref[0]
    gm_scr[k] = jnp.mean(jax.nn.gelu(h), axis=0, keepdims=True)

    @pl.when(k == _B - 1)
    def _last():
        _run_finalize(gm_scr, acc_ref, b2_ref, w2buf[par],
                      cs_ref[k], _B, es_ref[k])
        p = acc_ref[...]
        mu = jnp.mean(p, axis=0, keepdims=True)
        d = p - mu
        var = jnp.mean(d * d, axis=0, keepdims=True)
        normed = g_ref[...] * d * lax.rsqrt(var + 1e-5) + bt_ref[...]
        for i in range(_B):
            out_ref[perm_ref[i]] = normed[i:i + 1, :]


def _make_gather_kernel():
    nw = _NC * _NS
    ntok = _B * _S
    per_w = ntok // nw
    ch = 32
    nch = per_w // ch
    mesh = plsc.VectorSubcoreMesh(core_axis_name="c", subcore_axis_name="s",
                                  num_cores=_NC, num_subcores=_NS)

    @functools.partial(
        pl.kernel,
        mesh=mesh,
        out_type=jax.ShapeDtypeStruct((ntok, _H), jnp.float32),
        scratch_types=[
            pltpu.VMEM((per_w,), jnp.int32),
            pltpu.VMEM((2, ch, _H), jnp.float32),
            pltpu.SemaphoreType.DMA,
            pltpu.SemaphoreType.DMA,
            pltpu.SemaphoreType.DMA,
            pltpu.SemaphoreType.DMA,
        ],
    )
    def _gather(table_hbm, ids_hbm, out_hbm, idx_v, rows_v, g0, g1, t0, t1):
        wid = lax.axis_index("s") * _NC + lax.axis_index("c")
        base = wid * per_w
        gs = (g0, g1)
        ts = (t0, t1)
        pltpu.sync_copy(ids_hbm.at[pl.ds(base, per_w)], idx_v)

        def gat(c, b):
            return pltpu.make_async_copy(
                table_hbm.at[idx_v.at[pl.ds(c * ch, ch)]], rows_v.at[b], gs[b])

        def sca(c, b):
            return pltpu.make_async_copy(
                rows_v.at[b], out_hbm.at[pl.ds(base + c * ch, ch)], ts[b])

        gat(0, 0).start()
        for c in range(nch):
            b = c & 1
            gat(c, b).wait()
            if c + 1 < nch:
                nb = (c + 1) & 1
                if c + 1 >= 2:
                    sca(c - 1, nb).wait()
                gat(c + 1, nb).start()
            sca(c, b).start()
        sca(nch - 2, 0 if (nch - 2) % 2 == 0 else 1).wait()
        sca(nch - 1, 0 if (nch - 1) % 2 == 0 else 1).wait()

    return _gather


_gather_cache = []


def _gather_rows(table, ids):
    if not _gather_cache:
        _gather_cache.append(_make_gather_kernel())
    return _gather_cache[0](table, ids)


def kernel(router_inputs, input_ids, W_router, b_router, embed_table, W1, b1, W2, b2, bn_gamma, bn_beta):
    routed = pl.pallas_call(
        _router_body,
        out_shape=[jax.ShapeDtypeStruct((_B, 1), jnp.int32)] * 8,
    )(router_inputs, W_router, W_router.T,
      b_router.reshape(1, _E), b_router.reshape(_E, 1))
    perm, es, start, ordn, nxe, hn, ps, cs = (r.reshape(_B) for r in routed)

    x = _gather_rows(embed_table, input_ids.reshape(-1))
    x = x.reshape(_B, _S, _H)

    pred = pl.pallas_call(
        _moe_body,
        grid_spec=pltpu.PrefetchScalarGridSpec(
            num_scalar_prefetch=8,
            grid=(_B,),
            in_specs=[
                pl.BlockSpec((1, _S, _H),
                             lambda k, p, e, s, o, nx, h, pp, cc: (p[k], 0, 0)),
                pl.BlockSpec((1, 1, _F),
                             lambda k, p, e, s, o, nx, h, pp, cc: (e[k], 0, 0)),
                pl.BlockSpec((_E, 1, _H),
                             lambda k, p, e, s, o, nx, h, pp, cc: (0, 0, 0)),
                pl.BlockSpec((1, _H),
                             lambda k, p, e, s, o, nx, h, pp, cc: (0, 0)),
                pl.BlockSpec((1, _H),
                             lambda k, p, e, s, o, nx, h, pp, cc: (0, 0)),
                pl.BlockSpec(memory_space=pl.ANY),
                pl.BlockSpec(memory_space=pl.ANY),
            ],
            out_specs=pl.BlockSpec((_B, 1, _H),
                                   lambda k, p, e, s, o, nx, h, pp, cc: (0, 0, 0)),
            scratch_shapes=[
                pltpu.VMEM((2, _H, _F), jnp.float32),
                pltpu.VMEM((2, _F, _H), jnp.float32),
                pltpu.VMEM((_B, 1, _F), jnp.float32),
                pltpu.VMEM((_B, _H), jnp.float32),
                pltpu.SemaphoreType.DMA,
                pltpu.SemaphoreType.DMA,
                pltpu.SemaphoreType.DMA,
                pltpu.SemaphoreType.DMA,
            ],
        ),
        out_shape=jax.ShapeDtypeStruct((_B, 1, _H), jnp.float32),
    )(perm, es, start, ordn, nxe, hn, ps, cs,
      x, b1.reshape(_E, 1, _F), b2.reshape(_E, 1, _H),
      bn_gamma.reshape(1, _H), bn_beta.reshape(1, _H), W1, W2)
    return pred.reshape(_B, _H)


# speedup vs baseline: 5.7746x; 1.0104x over previous
import functools

import jax
import jax.numpy as jnp
from jax import lax
from jax.experimental import pallas as pl
from jax.experimental.pallas import tpu as pltpu
from jax.experimental.pallas import tpu_sc as plsc

_B, _S, _H, _V, _E, _F = 64, 128, 1024, 32000, 8, 2048
_CH = 64
_NC, _NS = 2, 16


def _fiota(shape, dim):
    return lax.broadcasted_iota(jnp.int32, shape, dim).astype(jnp.float32)


def _nt(a, b):
    return lax.dot_general(a, b, (((1,), (1,)), ((), ())),
                           preferred_element_type=jnp.float32)


def _router_body(x_ref, w_ref, wt_ref, b_ref, bt_ref,
                 perm_ref, es_ref, start_ref, ord_ref, nxe_ref, hn_ref,
                 ps_ref, cs_ref):
    x = x_ref[...]
    logits = jnp.dot(x, w_ref[...], preferred_element_type=jnp.float32) + b_ref[...]
    logits_t = _nt(wt_ref[...], x) + bt_ref[...]
    col = _fiota((_B, _E), 1)
    mx = jnp.max(logits, axis=1, keepdims=True)
    chosen_c = jnp.min(jnp.where(logits >= mx, col, float(_E)), axis=1, keepdims=True)
    row = _fiota((_E, _B), 0)
    mx_t = jnp.max(logits_t, axis=0, keepdims=True)
    chosen_r = jnp.min(jnp.where(logits_t >= mx_t, row, float(_E)), axis=0, keepdims=True)
    ic = _fiota((_B, 1), 0)
    ir = _fiota((1, _B), 1)
    key_c = chosen_c * _B + ic
    key_r = chosen_r * _B + ir
    rank_r = jnp.sum(jnp.where(key_c < key_r, 1.0, 0.0), axis=0, keepdims=True)
    rank_c = jnp.sum(jnp.where(key_r < key_c, 1.0, 0.0), axis=1, keepdims=True)
    rr = _fiota((_B, _B), 0)
    cc = _fiota((_B, _B), 1)
    r_mat = jnp.where(jnp.broadcast_to(rank_r, (_B, _B)) == rr, 1.0, 0.0)
    perm_c = _nt(r_mat, ir)
    es_c = _nt(r_mat, chosen_r)
    q_mat = jnp.where(jnp.broadcast_to(rank_c, (_B, _B)) == cc, 1.0, 0.0)
    es_row = jnp.dot(chosen_r, q_mat, preferred_element_type=jnp.float32)
    shift = jnp.where(rr == cc + 1.0, 1.0, 0.0)
    prev_c = jnp.dot(shift, es_c + 1.0, preferred_element_type=jnp.float32)
    start_c = jnp.where(es_c + 1.0 != prev_c, 1.0, 0.0)
    shift_r = jnp.where(cc == rr + 1.0, 1.0, 0.0)
    prev_r = jnp.dot(es_row + 1.0, shift_r, preferred_element_type=jnp.float32)
    start_row = jnp.where(es_row + 1.0 != prev_r, 1.0, 0.0)
    lincl = jnp.where(rr >= cc, 1.0, 0.0)
    ord_c = jnp.dot(lincl, start_c, preferred_element_type=jnp.float32) - 1.0
    lincl_r = jnp.where(cc >= rr, 1.0, 0.0)
    ord_row = jnp.dot(start_row, lincl_r, preferred_element_type=jnp.float32) - 1.0
    n_mat = jnp.where(
        jnp.logical_and(jnp.broadcast_to(ord_row, (_B, _B)) == ord_c + 1.0,
                        jnp.broadcast_to(start_row, (_B, _B)) > 0.5),
        1.0, 0.0)
    nxe_c = _nt(n_mat, es_row)
    hn_c = jnp.sum(n_mat, axis=1, keepdims=True)
    p_mat = jnp.where(
        jnp.logical_and(jnp.broadcast_to(ord_row, (_B, _B)) == ord_c - 1.0,
                        jnp.broadcast_to(start_row, (_B, _B)) > 0.5),
        1.0, 0.0)
    c_mat = jnp.where(
        jnp.logical_and(jnp.broadcast_to(ord_row, (_B, _B)) == ord_c,
                        jnp.broadcast_to(start_row, (_B, _B)) > 0.5),
        1.0, 0.0)
    ps_c = _nt(p_mat, ir)
    cs_c = _nt(c_mat, ir)
    perm_ref[...] = perm_c.astype(jnp.int32)
    es_ref[...] = es_c.astype(jnp.int32)
    start_ref[...] = start_c.astype(jnp.int32)
    ord_ref[...] = ord_c.astype(jnp.int32)
    nxe_ref[...] = nxe_c.astype(jnp.int32)
    hn_ref[...] = hn_c.astype(jnp.int32)
    ps_ref[...] = ps_c.astype(jnp.int32)
    cs_ref[...] = cs_c.astype(jnp.int32)


def _w_copies(w1_hbm, w2_hbm, e_idx, w1buf, w2buf, buf, sem1, sem2):
    c1 = pltpu.make_async_copy(w1_hbm.at[e_idx], w1buf.at[buf], sem1)
    c2 = pltpu.make_async_copy(w2_hbm.at[e_idx], w2buf.at[buf], sem2)
    return c1, c2


def _run_finalize(gm_scr, acc_ref, b2_ref, w2_e, lo, hi, e_idx):
    rows = lax.broadcasted_iota(jnp.int32, (_B, 1), 0)
    mask = jnp.logical_and(rows >= lo, rows < hi)
    gmm = jnp.where(mask, gm_scr[...].reshape(_B, _F), 0.0)
    contrib = jnp.dot(gmm, w2_e, preferred_element_type=jnp.float32)
    bias = jnp.where(mask, b2_ref[e_idx], 0.0)
    acc_ref[...] = acc_ref[...] + contrib + bias


def _moe_body(perm_ref, es_ref, start_ref, ord_ref, nxe_ref, hn_ref, ps_ref, cs_ref,
              x_ref, b1_ref, b2_ref, g_ref, bt_ref, w1_hbm, w2_hbm, out_ref,
              w1buf, w2buf, w1bf, gm_scr, acc_ref, s1a, s2a, s1b, s2b):
    k = pl.program_id(0)
    par = ord_ref[k] % 2
    is_start = start_ref[k] == 1

    @pl.when(k == 0)
    def _prime():
        acc_ref[...] = jnp.zeros_like(acc_ref)
        c1, c2 = _w_copies(w1_hbm, w2_hbm, es_ref[0], w1buf, w2buf, 0, s1a, s2a)
        c1.start()
        c2.start()

    @pl.when(jnp.logical_and(is_start, par == 0))
    def _flip0():
        @pl.when(k > 0)
        def _fin():
            _run_finalize(gm_scr, acc_ref, b2_ref, w2buf[1],
                          ps_ref[k], k, es_ref[k - 1])

        c1, c2 = _w_copies(w1_hbm, w2_hbm, es_ref[k], w1buf, w2buf, 0, s1a, s2a)
        c1.wait()
        c2.wait()
        w1bf[0] = w1buf[0].astype(jnp.bfloat16)

        @pl.when(hn_ref[k] == 1)
        def _next():
            n1, n2 = _w_copies(w1_hbm, w2_hbm, nxe_ref[k], w1buf, w2buf, 1, s1b, s2b)
            n1.start()
            n2.start()

    @pl.when(jnp.logical_and(is_start, par == 1))
    def _flip1():
        @pl.when(k > 0)
        def _fin():
            _run_finalize(gm_scr, acc_ref, b2_ref, w2buf[0],
                          ps_ref[k], k, es_ref[k - 1])

        c1, c2 = _w_copies(w1_hbm, w2_hbm, es_ref[k], w1buf, w2buf, 1, s1b, s2b)
        c1.wait()
        c2.wait()
        w1bf[1] = w1buf[1].astype(jnp.bfloat16)

        @pl.when(hn_ref[k] == 1)
        def _next():
            n1, n2 = _w_copies(w1_hbm, w2_hbm, nxe_ref[k], w1buf, w2buf, 0, s1a, s2a)
            n1.start()
            n2.start()

    xb = x_ref[0].astype(jnp.bfloat16)
    h = jnp.dot(xb, w1bf[par], preferred_element_type=jnp.float32) + b1_ref[0]
    gm_scr[k] = jnp.mean(jax.nn.gelu(h), axis=0, keepdims=True)

    @pl.when(k == _B - 1)
    def _last():
        _run_finalize(gm_scr, acc_ref, b2_ref, w2buf[par],
                      cs_ref[k], _B, es_ref[k])
        p = acc_ref[...]
        mu = jnp.mean(p, axis=0, keepdims=True)
        d = p - mu
        var = jnp.mean(d * d, axis=0, keepdims=True)
        normed = g_ref[...] * d * lax.rsqrt(var + 1e-5) + bt_ref[...]
        for i in range(_B):
            out_ref[perm_ref[i]] = normed[i:i + 1, :]


def _make_gather_kernel():
    nw = _NC * _NS
    ntok = _B * _S
    per_w = ntok // nw
    ch = 32
    nch = per_w // ch
    mesh = plsc.VectorSubcoreMesh(core_axis_name="c", subcore_axis_name="s",
                                  num_cores=_NC, num_subcores=_NS)

    @functools.partial(
        pl.kernel,
        mesh=mesh,
        out_type=jax.ShapeDtypeStruct((ntok, _H), jnp.float32),
        scratch_types=[
            pltpu.VMEM((per_w,), jnp.int32),
            pltpu.VMEM((2, ch, _H), jnp.float32),
            pltpu.SemaphoreType.DMA,
            pltpu.SemaphoreType.DMA,
            pltpu.SemaphoreType.DMA,
            pltpu.SemaphoreType.DMA,
        ],
    )
    def _gather(table_hbm, ids_hbm, out_hbm, idx_v, rows_v, g0, g1, t0, t1):
        wid = lax.axis_index("s") * _NC + lax.axis_index("c")
        base = wid * per_w
        gs = (g0, g1)
        ts = (t0, t1)
        pltpu.sync_copy(ids_hbm.at[pl.ds(base, per_w)], idx_v)

        def gat(c, b):
            return pltpu.make_async_copy(
                table_hbm.at[idx_v.at[pl.ds(c * ch, ch)]], rows_v.at[b], gs[b])

        def sca(c, b):
            return pltpu.make_async_copy(
                rows_v.at[b], out_hbm.at[pl.ds(base + c * ch, ch)], ts[b])

        gat(0, 0).start()
        for c in range(nch):
            b = c & 1
            gat(c, b).wait()
            if c + 1 < nch:
                nb = (c + 1) & 1
                if c + 1 >= 2:
                    sca(c - 1, nb).wait()
                gat(c + 1, nb).start()
            sca(c, b).start()
        sca(nch - 2, 0 if (nch - 2) % 2 == 0 else 1).wait()
        sca(nch - 1, 0 if (nch - 1) % 2 == 0 else 1).wait()

    return _gather


_gather_cache = []


def _gather_rows(table, ids):
    if not _gather_cache:
        _gather_cache.append(_make_gather_kernel())
    return _gather_cache[0](table, ids)


def kernel(router_inputs, input_ids, W_router, b_router, embed_table, W1, b1, W2, b2, bn_gamma, bn_beta):
    routed = pl.pallas_call(
        _router_body,
        out_shape=[jax.ShapeDtypeStruct((_B, 1), jnp.int32)] * 8,
    )(router_inputs, W_router, W_router.T,
      b_router.reshape(1, _E), b_router.reshape(_E, 1))
    perm, es, start, ordn, nxe, hn, ps, cs = (r.reshape(_B) for r in routed)

    x = _gather_rows(embed_table, input_ids.reshape(-1))
    x = x.reshape(_B, _S, _H)

    pred = pl.pallas_call(
        _moe_body,
        grid_spec=pltpu.PrefetchScalarGridSpec(
            num_scalar_prefetch=8,
            grid=(_B,),
            in_specs=[
                pl.BlockSpec((1, _S, _H),
                             lambda k, p, e, s, o, nx, h, pp, cc: (p[k], 0, 0)),
                pl.BlockSpec((1, 1, _F),
                             lambda k, p, e, s, o, nx, h, pp, cc: (e[k], 0, 0)),
                pl.BlockSpec((_E, 1, _H),
                             lambda k, p, e, s, o, nx, h, pp, cc: (0, 0, 0)),
                pl.BlockSpec((1, _H),
                             lambda k, p, e, s, o, nx, h, pp, cc: (0, 0)),
                pl.BlockSpec((1, _H),
                             lambda k, p, e, s, o, nx, h, pp, cc: (0, 0)),
                pl.BlockSpec(memory_space=pl.ANY),
                pl.BlockSpec(memory_space=pl.ANY),
            ],
            out_specs=pl.BlockSpec((_B, 1, _H),
                                   lambda k, p, e, s, o, nx, h, pp, cc: (0, 0, 0)),
            scratch_shapes=[
                pltpu.VMEM((2, _H, _F), jnp.float32),
                pltpu.VMEM((2, _F, _H), jnp.float32),
                pltpu.VMEM((2, _H, _F), jnp.bfloat16),
                pltpu.VMEM((_B, 1, _F), jnp.float32),
                pltpu.VMEM((_B, _H), jnp.float32),
                pltpu.SemaphoreType.DMA,
                pltpu.SemaphoreType.DMA,
                pltpu.SemaphoreType.DMA,
                pltpu.SemaphoreType.DMA,
            ],
        ),
        out_shape=jax.ShapeDtypeStruct((_B, 1, _H), jnp.float32),
    )(perm, es, start, ordn, nxe, hn, ps, cs,
      x, b1.reshape(_E, 1, _F), b2.reshape(_E, 1, _H),
      bn_gamma.reshape(1, _H), bn_beta.reshape(1, _H), W1, W2)
    return pred.reshape(_B, _H)
